# baseline (speedup 1.0000x reference)
import functools

import jax
import jax.numpy as jnp
from jax import lax
from jax.experimental import pallas as pl
from jax.experimental.pallas import tpu as pltpu
from jax.experimental.pallas import tpu_sc as plsc

B = 16384
ROWS = (1, 5, 3, 2)
WORDS = (2, 3, 1, 1)
NC, NS, L = 2, 16, 16
NW = NC * NS
NB = B // NW
CHUNK = 128
NG = CHUNK // 8
NCHUNK = NB // CHUNK


def _sc_body(x_hbm, out_hbm, buf, outv, sem):
    wid = lax.axis_index("s") * NC + lax.axis_index("c")
    base = wid * NB

    j = lax.iota(jnp.int32, L)
    i_vec = j & 3
    q_vec = j >> 2
    w_vec = jnp.where(i_vec == 0, 2, jnp.where(i_vec == 1, 3, 1))

    for chunk in range(NCHUNK):
        gbase = (base + chunk * CHUNK) // 8
        descs = [
            pltpu.async_copy(
                x_hbm.at[pl.ds(gbase, NG), :, ROWS[i], :], buf.at[i], sem)
            for i in range(4)
        ]
        for d in descs:
            d.wait()

        def body(c, _):
            b_vec = q_vec + c * 4
            val = plsc.load_gather(
                buf, [i_vec, b_vec >> 3, b_vec & 7, w_vec])
            outv[pl.ds(chunk * CHUNK * 4 + c * L, L)] = val
            return _

        lax.fori_loop(0, (CHUNK * 4) // L, body, None)

    pltpu.sync_copy(outv, out_hbm.at[pl.ds(base * 4, NB * 4)])


@jax.jit
def _sc_call(x4):
    mesh = plsc.VectorSubcoreMesh(
        core_axis_name="c", subcore_axis_name="s",
        num_cores=NC, num_subcores=NS)
    flat = pl.kernel(
        _sc_body,
        out_type=jax.ShapeDtypeStruct((B * 4,), jnp.float32),
        mesh=mesh,
        scratch_types=[
            pltpu.VMEM((4, NG, 8, 128), jnp.float32),
            pltpu.VMEM((NB * 4,), jnp.float32),
            pltpu.SemaphoreType.DMA,
        ],
        compiler_params=pltpu.CompilerParams(needs_layout_passes=False),
    )(x4)
    return flat.reshape(B, 4)


def kernel(X):
    return _sc_call(X.reshape(B // 8, 8, 50, 128))
